# baseline (speedup 1.0000x reference)
import functools

import jax
import jax.numpy as jnp
from jax import lax
from jax.experimental import pallas as pl
from jax.experimental.pallas import tpu as pltpu
from jax.experimental.pallas import tpu_sc as plsc

D = 32
L = 16
TILE_C = 128
WAVE = 16


@functools.lru_cache(maxsize=None)
def _build_sc_kernel(b: int, nc: int, ns: int):
    nw = nc * ns
    b_per_w = b // nw
    n_waves = b_per_w // WAVE
    gi = b_per_w // TILE_C
    mesh = plsc.VectorSubcoreMesh(core_axis_name="c", subcore_axis_name="s")

    @functools.partial(
        pl.kernel,
        mesh=mesh,
        out_type=jax.ShapeDtypeStruct((nw, 8, TILE_C), jnp.float32),
        compiler_params=pltpu.CompilerParams(needs_layout_passes=False,
                                             use_tc_tiling_on_sc=True),
        scratch_types=[
            pltpu.VMEM((8, TILE_C), jnp.float32),
            pltpu.VMEM((2 * gi, TILE_C), jnp.int32),
            pltpu.VMEM((WAVE, D, TILE_C), jnp.float32),
            pltpu.VMEM((D, L), jnp.float32),
            pltpu.VMEM((D, L), jnp.float32),
            pltpu.VMEM((8, TILE_C), jnp.float32),
            pltpu.SemaphoreType.DMA,
            pltpu.SemaphoreType.DMA,
        ],
    )
    def sc_kernel(idx_hbm, sppmi_hbm, items_t_hbm, ctxs_t_hbm, out_hbm,
                  sppmi_v, idx_v, buf_v, iblk_v, cblk_v, res_v,
                  sem_a, sem_b):
        wid = lax.axis_index("s") * nc + lax.axis_index("c")
        pltpu.sync_copy(idx_hbm.at[wid], idx_v)
        pltpu.sync_copy(sppmi_hbm.at[wid], sppmi_v)

        lane = lax.iota(jnp.int32, L)
        dv0 = lane
        dv1 = lane + L

        def extract(slot, l_col, blk, i16):
            slot_v = jnp.full((L,), slot, jnp.int32)
            l_v = jnp.full((L,), l_col, jnp.int32)
            i_v = jnp.full((L,), i16, jnp.int32)
            v0 = plsc.load_gather(buf_v, [slot_v, dv0, l_v])
            v1 = plsc.load_gather(buf_v, [slot_v, dv1, l_v])
            plsc.store_scatter(blk, [dv0, i_v], v0)
            plsc.store_scatter(blk, [dv1, i_v], v1)

        def fetch_extract(table_hbm, jvec, blk):
            copies = []
            for u in range(WAVE):
                cj = pl.multiple_of((jvec[u] // TILE_C) * TILE_C, TILE_C)
                for r in range(D // 8):
                    copies.append(pltpu.async_copy(
                        table_hbm.at[pl.ds(8 * r, 8), pl.ds(cj, TILE_C)],
                        buf_v.at[u, pl.ds(8 * r, 8)], sem_a))
            for cp in copies:
                cp.wait()
            for u in range(WAVE):
                extract(u, jvec[u] % TILE_C, blk, u)

        def wave_body(w, acc):
            g = w // (TILE_C // WAVE)
            k = (w * WAVE) % TILE_C
            jvec_i = idx_v[g, pl.ds(k, L)]
            jvec_c = idx_v[gi + g, pl.ds(k, L)]
            fetch_extract(items_t_hbm, jvec_i, iblk_v)
            fetch_extract(ctxs_t_hbm, jvec_c, cblk_v)

            pred = jnp.zeros((L,), jnp.float32)
            for d in range(D):
                pred = pred + iblk_v[d, :] * cblk_v[d, :]
            s = sppmi_v[g, pl.ds(k, L)]
            diff = s - pred
            res_v[0, pl.ds(0, L)] = res_v[0, pl.ds(0, L)] + diff * diff
            return acc

        for k in range(8):
            for m in range(TILE_C // L):
                res_v[k, pl.ds(m * L, L)] = jnp.zeros((L,), jnp.float32)
        lax.fori_loop(0, n_waves, wave_body, 0)
        pltpu.sync_copy(res_v, out_hbm.at[wid])

    return sc_kernel


def kernel(user_id, item_id, rating, users, items, contexts):
    del users
    b = user_id.shape[0]
    info = plsc.get_sparse_core_info()
    nc, ns = info.num_cores, info.num_subcores
    nw = nc * ns
    bw = b // nw
    gi = bw // TILE_C
    item_idx = user_id.astype(jnp.int32).reshape(nw, gi, TILE_C)
    ctx_idx = item_id.astype(jnp.int32).reshape(nw, gi, TILE_C)
    idx = jnp.concatenate([item_idx, ctx_idx], axis=1)
    sppmi = jnp.pad(rating.astype(jnp.float32).reshape(nw, gi, TILE_C),
                    ((0, 0), (0, 8 - gi), (0, 0)))
    partial = _build_sc_kernel(b, nc, ns)(idx, sppmi, items.T, contexts.T)
    return jnp.sum(partial) / b


# speedup vs baseline: 1.0054x; 1.0054x over previous
import functools

import jax
import jax.numpy as jnp
from jax import lax
from jax.experimental import pallas as pl
from jax.experimental.pallas import tpu as pltpu
from jax.experimental.pallas import tpu_sc as plsc

D = 32
L = 16
TILE_C = 128
WAVE = 16


@functools.lru_cache(maxsize=None)
def _build_sc_kernel(b: int, nc: int, ns: int):
    nw = nc * ns
    b_per_w = b // nw
    n_waves = b_per_w // WAVE
    gi = b_per_w // TILE_C
    mesh = plsc.VectorSubcoreMesh(core_axis_name="c", subcore_axis_name="s")

    @functools.partial(
        pl.kernel,
        mesh=mesh,
        out_type=jax.ShapeDtypeStruct((nw, 8, TILE_C), jnp.float32),
        compiler_params=pltpu.CompilerParams(needs_layout_passes=False,
                                             use_tc_tiling_on_sc=True),
        scratch_types=[
            pltpu.VMEM((8, TILE_C), jnp.float32),
            pltpu.VMEM((2 * gi, TILE_C), jnp.int32),
            pltpu.VMEM((WAVE, D, TILE_C), jnp.float32),
            pltpu.VMEM((D, L), jnp.float32),
            pltpu.VMEM((D, L), jnp.float32),
            pltpu.VMEM((8, TILE_C), jnp.float32),
            pltpu.SemaphoreType.DMA,
            pltpu.SemaphoreType.DMA,
        ],
    )
    def sc_kernel(idx_hbm, sppmi_hbm, items_t_hbm, ctxs_t_hbm, out_hbm,
                  sppmi_v, idx_v, buf_v, iblk_v, cblk_v, res_v,
                  sem_a, sem_b):
        wid = lax.axis_index("s") * nc + lax.axis_index("c")
        pltpu.sync_copy(idx_hbm.at[wid], idx_v)
        pltpu.sync_copy(sppmi_hbm.at[wid], sppmi_v)

        lane = lax.iota(jnp.int32, L)
        dv0 = lane
        dv1 = lane + L

        def extract(slot, l_col, blk, i16):
            slot_v = jnp.full((L,), slot, jnp.int32)
            l_v = jnp.full((L,), l_col, jnp.int32)
            i_v = jnp.full((L,), i16, jnp.int32)
            v0 = plsc.load_gather(buf_v, [slot_v, dv0, l_v])
            v1 = plsc.load_gather(buf_v, [slot_v, dv1, l_v])
            plsc.store_scatter(blk, [dv0, i_v], v0)
            plsc.store_scatter(blk, [dv1, i_v], v1)

        def fetch_extract(table_hbm, jvec, blk):
            copies = []
            for u in range(WAVE):
                cj = pl.multiple_of((jvec[u] // TILE_C) * TILE_C, TILE_C)
                copies.append(pltpu.async_copy(
                    table_hbm.at[:, pl.ds(cj, TILE_C)], buf_v.at[u], sem_a))
            for cp in copies:
                cp.wait()
            for u in range(WAVE):
                extract(u, jvec[u] % TILE_C, blk, u)

        def wave_body(w, acc):
            g = w // (TILE_C // WAVE)
            k = (w * WAVE) % TILE_C
            jvec_i = idx_v[g, pl.ds(k, L)]
            jvec_c = idx_v[gi + g, pl.ds(k, L)]
            fetch_extract(items_t_hbm, jvec_i, iblk_v)
            fetch_extract(ctxs_t_hbm, jvec_c, cblk_v)

            pred = jnp.zeros((L,), jnp.float32)
            for d in range(D):
                pred = pred + iblk_v[d, :] * cblk_v[d, :]
            s = sppmi_v[g, pl.ds(k, L)]
            diff = s - pred
            res_v[0, pl.ds(0, L)] = res_v[0, pl.ds(0, L)] + diff * diff
            return acc

        for k in range(8):
            for m in range(TILE_C // L):
                res_v[k, pl.ds(m * L, L)] = jnp.zeros((L,), jnp.float32)
        lax.fori_loop(0, n_waves, wave_body, 0)
        pltpu.sync_copy(res_v, out_hbm.at[wid])

    return sc_kernel


def kernel(user_id, item_id, rating, users, items, contexts):
    del users
    b = user_id.shape[0]
    info = plsc.get_sparse_core_info()
    nc, ns = info.num_cores, info.num_subcores
    nw = nc * ns
    bw = b // nw
    gi = bw // TILE_C
    item_idx = user_id.astype(jnp.int32).reshape(nw, gi, TILE_C)
    ctx_idx = item_id.astype(jnp.int32).reshape(nw, gi, TILE_C)
    idx = jnp.concatenate([item_idx, ctx_idx], axis=1)
    sppmi = jnp.pad(rating.astype(jnp.float32).reshape(nw, gi, TILE_C),
                    ((0, 0), (0, 8 - gi), (0, 0)))
    partial = _build_sc_kernel(b, nc, ns)(idx, sppmi, items.T, contexts.T)
    return jnp.sum(partial) / b
